# baseline (speedup 1.0000x reference)
import jax
import jax.numpy as jnp
from jax import lax
from jax.experimental import pallas as pl
from jax.experimental.pallas import tpu as pltpu
from jax.experimental.pallas import tpu_sc as plsc

N_NODES = 10000
N_EDGES = 320000
D = 128

NC = 2
NS = 16
NW = NC * NS
E_PER_W = N_EDGES // NW
CHUNK = 80
N_CHUNKS = 126
E_PAD_W = N_CHUNKS * CHUNK
N_PAD = 10240
ROWS_PER_TILE = N_PAD // NS


def _sc_body(x_hbm, src_hbm, dst_hbm,
             part_hbm, cnt_hbm,
             acc_sh, cnt_sh, src_v, dst_v, rows0, rows1, ones_v,
             gsem0, gsem1):
  cid = lax.axis_index("c")
  sid = lax.axis_index("s")
  wid = cid * NS + sid

  def zfill(r, carry):
    for k in range(D // 16):
      rows0[r, pl.ds(k * 16, 16)] = jnp.zeros((16,), jnp.float32)
    return carry

  lax.fori_loop(0, CHUNK, zfill, 0)
  base = sid * ROWS_PER_TILE
  for c in range(ROWS_PER_TILE // CHUNK):
    pltpu.sync_copy(rows0, acc_sh.at[pl.ds(base + c * CHUNK, CHUNK)])
  for c in range(ROWS_PER_TILE // CHUNK):
    pltpu.sync_copy(rows0.at[0, pl.ds(0, CHUNK)],
                    cnt_sh.at[pl.ds(base + c * CHUNK, CHUNK)])

  pltpu.sync_copy(src_hbm.at[wid], src_v)
  pltpu.sync_copy(dst_hbm.at[wid], dst_v)

  for k in range(CHUNK // 16):
    ones_v[pl.ds(k * 16, 16)] = jnp.full((16,), 1.0, dtype=jnp.float32)

  plsc.subcore_barrier()

  def step(t, carry):
    jj = 2 * t
    g0 = pltpu.async_copy(x_hbm.at[src_v.at[pl.ds(jj * CHUNK, CHUNK)]],
                          rows0, gsem0)
    pltpu.sync_copy(ones_v, cnt_sh.at[dst_v.at[jj]], add=True)
    g0.wait()
    g1 = pltpu.async_copy(
        x_hbm.at[src_v.at[pl.ds((jj + 1) * CHUNK, CHUNK)]], rows1, gsem1)
    pltpu.sync_copy(rows0, acc_sh.at[dst_v.at[jj]], add=True)
    pltpu.sync_copy(ones_v, cnt_sh.at[dst_v.at[jj + 1]], add=True)
    g1.wait()
    pltpu.sync_copy(rows1, acc_sh.at[dst_v.at[jj + 1]], add=True)
    return carry

  lax.fori_loop(0, N_CHUNKS // 2, step, 0)

  plsc.subcore_barrier()

  pending = []
  for c in range(ROWS_PER_TILE // CHUNK):
    buf, sem = (rows0, gsem0) if c % 2 == 0 else (rows1, gsem1)
    if c >= 2:
      pending[c - 2].wait()
    pltpu.sync_copy(acc_sh.at[pl.ds(base + c * CHUNK, CHUNK)], buf)
    pending.append(
        pltpu.async_copy(buf, part_hbm.at[cid, pl.ds(base + c * CHUNK, CHUNK)],
                         sem))
  pending[-2].wait()
  pending[-1].wait()

  @pl.when(sid == 0)
  def _():
    pltpu.sync_copy(cnt_sh, cnt_hbm.at[cid])


@jax.jit
def _sc_segment_sum(x, src3, dst3):
  mesh = plsc.VectorSubcoreMesh(core_axis_name="c", subcore_axis_name="s")
  k = pl.kernel(
      _sc_body,
      out_type=[
          jax.ShapeDtypeStruct((NC, N_PAD, D), jnp.float32),
          jax.ShapeDtypeStruct((NC, N_PAD), jnp.float32),
      ],
      mesh=mesh,
      scratch_types=[
          pltpu.VMEM_SHARED((N_PAD, D), jnp.float32),
          pltpu.VMEM_SHARED((N_PAD,), jnp.float32),
          pltpu.VMEM((E_PAD_W,), jnp.int32),
          pltpu.VMEM((N_CHUNKS, CHUNK), jnp.int32),
          pltpu.VMEM((CHUNK, D), jnp.float32),
          pltpu.VMEM((CHUNK, D), jnp.float32),
          pltpu.VMEM((CHUNK,), jnp.float32),
          pltpu.SemaphoreType.DMA,
          pltpu.SemaphoreType.DMA,
      ],
  )
  return k(x, src3, dst3)


def _tc_body(part_ref, cnt_ref, x_ref, wl_ref, bl_ref, wr_ref, out_ref):
  summed = part_ref[0] + part_ref[1]
  counts = cnt_ref[0] + cnt_ref[1]
  mean = summed * (1.0 / jnp.maximum(counts, 1.0))
  out_ref[...] = (
      jnp.dot(mean, wl_ref[...], preferred_element_type=jnp.float32)
      + jnp.dot(x_ref[...], wr_ref[...], preferred_element_type=jnp.float32)
      + bl_ref[...]
  )


@jax.jit
def _tc_combine(part, cnt, x, W_l, b_l, W_r):
  R = 1000
  grid = (N_NODES // R,)
  return pl.pallas_call(
      _tc_body,
      grid=grid,
      in_specs=[
          pl.BlockSpec((NC, R, D), lambda i: (0, i, 0)),
          pl.BlockSpec((NC, R, 1), lambda i: (0, i, 0)),
          pl.BlockSpec((R, D), lambda i: (i, 0)),
          pl.BlockSpec((D, D), lambda i: (0, 0)),
          pl.BlockSpec((1, D), lambda i: (0, 0)),
          pl.BlockSpec((D, D), lambda i: (0, 0)),
      ],
      out_specs=pl.BlockSpec((R, D), lambda i: (i, 0)),
      out_shape=jax.ShapeDtypeStruct((N_NODES, D), jnp.float32),
  )(part, cnt.reshape(NC, N_PAD, 1), x, W_l, b_l.reshape(1, D), W_r)


def kernel(x, edge_index, W_l, b_l, W_r):
  ei = edge_index.astype(jnp.int32).reshape(2, NW, E_PER_W)
  pad = E_PAD_W - E_PER_W
  src3 = jnp.pad(ei[0], ((0, 0), (0, pad)))
  dst3 = jnp.pad(ei[1], ((0, 0), (0, pad)),
                 constant_values=N_PAD - 1).reshape(NW, N_CHUNKS, CHUNK)
  part, cnt = _sc_segment_sum(x, src3, dst3)
  return _tc_combine(part, cnt, x, W_l, b_l, W_r)


# speedup vs baseline: 1.2079x; 1.2079x over previous
import jax
import jax.numpy as jnp
from jax import lax
from jax.experimental import pallas as pl
from jax.experimental.pallas import tpu as pltpu
from jax.experimental.pallas import tpu_sc as plsc

N_NODES = 10000
N_EDGES = 320000
D = 128

NC = 2
NS = 16
NW = NC * NS
E_PER_W = N_EDGES // NW
CHUNK = 80
N_CHUNKS = E_PER_W // CHUNK
N_PAD = 10240
ROWS_PER_TILE = N_PAD // NS


def _sc_body(x_hbm, src_hbm, dst_hbm, zrow_hbm, zcnt_hbm,
             part_hbm, cnt_hbm,
             acc_sh, cnt_sh, src_v, dst_v, rows_v, ones_v, gsem):
  cid = lax.axis_index("c")
  sid = lax.axis_index("s")
  wid = cid * NS + sid

  pltpu.sync_copy(zrow_hbm.at[pl.ds(sid * ROWS_PER_TILE, ROWS_PER_TILE)],
                  acc_sh.at[pl.ds(sid * ROWS_PER_TILE, ROWS_PER_TILE)])

  @pl.when(sid == 0)
  def _():
    pltpu.sync_copy(zcnt_hbm, cnt_sh)

  pltpu.sync_copy(src_hbm.at[wid], src_v)
  pltpu.sync_copy(dst_hbm.at[wid], dst_v)

  for k in range(CHUNK // 16):
    ones_v[pl.ds(k * 16, 16)] = jnp.full((16,), 1.0, dtype=jnp.float32)

  plsc.subcore_barrier()

  def chunk_step(j, carry):
    g = pltpu.async_copy(x_hbm.at[src_v.at[j]], rows_v, gsem)
    pltpu.sync_copy(ones_v, cnt_sh.at[dst_v.at[j]], add=True)
    g.wait()
    pltpu.sync_copy(rows_v, acc_sh.at[dst_v.at[j]], add=True)
    return carry

  lax.fori_loop(0, N_CHUNKS, chunk_step, 0)

  plsc.subcore_barrier()

  pltpu.sync_copy(acc_sh.at[pl.ds(sid * ROWS_PER_TILE, ROWS_PER_TILE)],
                  part_hbm.at[cid, pl.ds(sid * ROWS_PER_TILE, ROWS_PER_TILE)])

  @pl.when(sid == 0)
  def _():
    pltpu.sync_copy(cnt_sh, cnt_hbm.at[cid])


@jax.jit
def _sc_segment_sum(x, src3, dst3):
  mesh = plsc.VectorSubcoreMesh(core_axis_name="c", subcore_axis_name="s")
  zrow = jnp.zeros((N_PAD, D), jnp.float32)
  zcnt = jnp.zeros((N_PAD,), jnp.float32)
  k = pl.kernel(
      _sc_body,
      out_type=[
          jax.ShapeDtypeStruct((NC, N_PAD, D), jnp.float32),
          jax.ShapeDtypeStruct((NC, N_PAD), jnp.float32),
      ],
      mesh=mesh,
      scratch_types=[
          pltpu.VMEM_SHARED((N_PAD, D), jnp.float32),
          pltpu.VMEM_SHARED((N_PAD,), jnp.float32),
          pltpu.VMEM((N_CHUNKS, CHUNK), jnp.int32),
          pltpu.VMEM((N_CHUNKS, CHUNK), jnp.int32),
          pltpu.VMEM((CHUNK, D), jnp.float32),
          pltpu.VMEM((CHUNK,), jnp.float32),
          pltpu.SemaphoreType.DMA,
      ],
  )
  return k(x, src3, dst3, zrow, zcnt)


def _tc_body(part_ref, cnt_ref, x_ref, wl_ref, bl_ref, wr_ref, out_ref):
  summed = part_ref[0] + part_ref[1]
  counts = cnt_ref[0] + cnt_ref[1]
  mean = summed * (1.0 / jnp.maximum(counts, 1.0))
  out_ref[...] = (
      jnp.dot(mean, wl_ref[...], preferred_element_type=jnp.float32)
      + jnp.dot(x_ref[...], wr_ref[...], preferred_element_type=jnp.float32)
      + bl_ref[...]
  )


@jax.jit
def _tc_combine(part, cnt, x, W_l, b_l, W_r):
  R = 1000
  grid = (N_NODES // R,)
  return pl.pallas_call(
      _tc_body,
      grid=grid,
      in_specs=[
          pl.BlockSpec((NC, R, D), lambda i: (0, i, 0)),
          pl.BlockSpec((NC, R, 1), lambda i: (0, i, 0)),
          pl.BlockSpec((R, D), lambda i: (i, 0)),
          pl.BlockSpec((D, D), lambda i: (0, 0)),
          pl.BlockSpec((1, D), lambda i: (0, 0)),
          pl.BlockSpec((D, D), lambda i: (0, 0)),
      ],
      out_specs=pl.BlockSpec((R, D), lambda i: (i, 0)),
      out_shape=jax.ShapeDtypeStruct((N_NODES, D), jnp.float32),
  )(part, cnt.reshape(NC, N_PAD, 1), x, W_l, b_l.reshape(1, D), W_r)


def kernel(x, edge_index, W_l, b_l, W_r):
  ei = edge_index.astype(jnp.int32).reshape(2, NW, N_CHUNKS, CHUNK)
  part, cnt = _sc_segment_sum(x, ei[0], ei[1])
  return _tc_combine(part, cnt, x, W_l, b_l, W_r)
